# baseline (speedup 1.0000x reference)
import functools

import jax
import jax.numpy as jnp
from jax import lax
from jax.experimental import pallas as pl
from jax.experimental.pallas import tpu as pltpu
from jax.experimental.pallas import tpu_sc as plsc

BLK = 128
PACK = 4
SUB_LOG = 14
SUB = 1 << SUB_LOG
GRP_LOG = SUB_LOG + 2
GRP = 1 << GRP_LOG


@functools.lru_cache(maxsize=None)
def _build(batch, hist, embed_dim, vocab):
    info = plsc.get_sparse_core_info()
    nc, ns = info.num_cores, info.num_subcores
    nw = nc * ns
    assert batch % BLK == 0 and (batch // BLK) == nw
    assert embed_dim == 32 and PACK * embed_dim == 128
    assert hist % 2 == 0
    n_grp = -(-vocab // GRP)

    mesh = plsc.VectorSubcoreMesh(core_axis_name="c", subcore_axis_name="s")

    @functools.partial(
        pl.kernel,
        mesh=mesh,
        out_type=jax.ShapeDtypeStruct((hist, embed_dim, batch), jnp.float32),
        scratch_types=[
            pltpu.VMEM((hist, BLK), jnp.int32),
            pltpu.VMEM((hist, BLK), jnp.int32),
            pltpu.VMEM((hist, BLK), jnp.int32),
            pltpu.VMEM((BLK, 128), jnp.float32),
            pltpu.VMEM((BLK, 128), jnp.float32),
            pltpu.VMEM((embed_dim, BLK), jnp.float32),
            pltpu.VMEM((embed_dim, BLK), jnp.float32),
            pltpu.SemaphoreType.DMA,
            pltpu.SemaphoreType.DMA,
        ],
        compiler_params=pltpu.CompilerParams(
            use_tc_tiling_on_sc=True, needs_layout_passes=False
        ),
    )
    def gather_kernel(xt_hbm, table_hbm, out_hbm, idx_v, idxq_v, colb_v,
                      rows_a, rows_b, t_a, t_b, gsem, osem):
        wid = lax.axis_index("s") * nc + lax.axis_index("c")
        b0 = wid * BLK
        pltpu.sync_copy(xt_hbm.at[:, pl.ds(b0, BLK)], idx_v)

        lane = lax.iota(jnp.int32, 16)

        @plsc.parallel_loop(0, hist)
        def _(h):
            for bq in range(BLK // 16):
                v = idx_v[h, pl.ds(bq * 16, 16)]
                r = lax.shift_right_logical(v, GRP_LOG)
                idxq_v[h, pl.ds(bq * 16, 16)] = (
                    lax.shift_left(r, SUB_LOG) | (v & (SUB - 1))
                )
                colb_v[h, pl.ds(bq * 16, 16)] = (
                    (lax.shift_right_logical(v, SUB_LOG) & (PACK - 1))
                    * embed_dim
                )

        def select_and_store(rows_ref, t_v, h):
            @plsc.parallel_loop(0, embed_dim, unroll=4)
            def _(e):
                for bq in range(BLK // 16):
                    c = colb_v[h, pl.ds(bq * 16, 16)] + e
                    r = lane + (bq * 16)
                    t_v[e, pl.ds(bq * 16, 16)] = plsc.load_gather(
                        rows_ref, [r, c]
                    )

            pltpu.async_copy(t_v, out_hbm.at[h, :, pl.ds(b0, BLK)], osem)

        def drain_out(t_v):
            pltpu.make_async_copy(t_v, out_hbm.at[0, :, pl.ds(b0, BLK)],
                                  osem).wait()

        def issue(h, dst_ref):
            return pltpu.async_copy(table_hbm.at[idxq_v.at[h]], dst_ref, gsem)

        def drain(dst_ref):
            pltpu.make_async_copy(table_hbm.at[idxq_v.at[0]], dst_ref,
                                  gsem).wait()

        issue(0, rows_a)

        def body(c, _):
            h0 = 2 * c
            issue(h0 + 1, rows_b)
            drain(rows_a)

            @pl.when(h0 >= 2)
            def _():
                drain_out(t_a)

            select_and_store(rows_a, t_a, h0)

            @pl.when(h0 + 2 < hist)
            def _():
                issue(h0 + 2, rows_a)

            drain(rows_b)

            @pl.when(h0 >= 2)
            def _():
                drain_out(t_b)

            select_and_store(rows_b, t_b, h0 + 1)
            return 0

        lax.fori_loop(0, hist // 2, body, 0)
        drain_out(t_a)
        drain_out(t_b)

    def repack_body(src_ref, dst_ref):
        x = src_ref[...]
        acc = None
        for k in range(PACK):
            sel = jnp.pad(
                jnp.eye(embed_dim, dtype=jnp.float32),
                ((0, 0), (k * embed_dim, (PACK - 1 - k) * embed_dim)),
            )
            part = jax.lax.dot_general(
                x[:, k * SUB:(k + 1) * SUB], sel,
                (((0,), (0,)), ((), ())),
                preferred_element_type=jnp.float32,
            )
            acc = part if acc is None else acc + part
        dst_ref[...] = acc

    repack = pl.pallas_call(
        repack_body,
        grid=(n_grp,),
        in_specs=[pl.BlockSpec((embed_dim, GRP), lambda j: (0, j))],
        out_specs=pl.BlockSpec((SUB, PACK * embed_dim), lambda j: (j, 0)),
        out_shape=jax.ShapeDtypeStruct((n_grp * SUB, PACK * embed_dim),
                                       jnp.float32),
        compiler_params=pltpu.CompilerParams(
            fuse_transposed_lhs_in_matmul=True
        ),
    )

    def run(x, raw_weight):
        xt = x.T.astype(jnp.int32)
        tq = repack(raw_weight.T)
        out_t = gather_kernel(xt, tq)
        return jnp.transpose(out_t, (2, 0, 1))

    return run


def kernel(x, raw_weight):
    batch, hist = x.shape
    vocab, embed_dim = raw_weight.shape
    return _build(batch, hist, embed_dim, vocab)(x, raw_weight)


# speedup vs baseline: 1.0126x; 1.0126x over previous
import functools

import jax
import jax.numpy as jnp
from jax import lax
from jax.experimental import pallas as pl
from jax.experimental.pallas import tpu as pltpu
from jax.experimental.pallas import tpu_sc as plsc

BLK = 128
PACK = 4
SUB_LOG = 13
SUB = 1 << SUB_LOG
GRP_LOG = SUB_LOG + 2
GRP = 1 << GRP_LOG


@functools.lru_cache(maxsize=None)
def _build(batch, hist, embed_dim, vocab):
    info = plsc.get_sparse_core_info()
    nc, ns = info.num_cores, info.num_subcores
    nw = nc * ns
    assert batch % BLK == 0 and (batch // BLK) == nw
    assert embed_dim == 32 and PACK * embed_dim == 128
    assert hist % 2 == 0
    n_grp = -(-vocab // GRP)

    mesh = plsc.VectorSubcoreMesh(core_axis_name="c", subcore_axis_name="s")

    @functools.partial(
        pl.kernel,
        mesh=mesh,
        out_type=jax.ShapeDtypeStruct((hist, embed_dim, batch), jnp.float32),
        scratch_types=[
            pltpu.VMEM((hist, BLK), jnp.int32),
            pltpu.VMEM((hist, BLK), jnp.int32),
            pltpu.VMEM((hist, BLK), jnp.int32),
            pltpu.VMEM((4, BLK, 128), jnp.float32),
            pltpu.VMEM((embed_dim, BLK), jnp.float32),
            pltpu.VMEM((embed_dim, BLK), jnp.float32),
            pltpu.SemaphoreType.DMA,
            pltpu.SemaphoreType.DMA,
        ],
        compiler_params=pltpu.CompilerParams(
            use_tc_tiling_on_sc=True, needs_layout_passes=False
        ),
    )
    def gather_kernel(xt_hbm, table_hbm, out_hbm, idx_v, idxq_v, colb_v,
                      rows_v, t_a, t_b, gsem, osem):
        wid = lax.axis_index("s") * nc + lax.axis_index("c")
        b0 = wid * BLK
        pltpu.sync_copy(xt_hbm.at[:, pl.ds(b0, BLK)], idx_v)

        lane = lax.iota(jnp.int32, 16)

        @plsc.parallel_loop(0, hist)
        def _(h):
            for bq in range(BLK // 16):
                v = idx_v[h, pl.ds(bq * 16, 16)]
                r = lax.shift_right_logical(v, GRP_LOG)
                idxq_v[h, pl.ds(bq * 16, 16)] = (
                    lax.shift_left(r, SUB_LOG) | (v & (SUB - 1))
                )
                colb_v[h, pl.ds(bq * 16, 16)] = (
                    (lax.shift_right_logical(v, SUB_LOG) & (PACK - 1))
                    * embed_dim
                )

        def select_and_store(rows_ref, t_v, h):
            @plsc.parallel_loop(0, embed_dim, unroll=4)
            def _(e):
                for bq in range(BLK // 16):
                    c = colb_v[h, pl.ds(bq * 16, 16)] + e
                    r = lane + (bq * 16)
                    t_v[e, pl.ds(bq * 16, 16)] = plsc.load_gather(
                        rows_ref, [r, c]
                    )

            pltpu.async_copy(t_v, out_hbm.at[h, :, pl.ds(b0, BLK)], osem)

        def drain_out(t_v):
            pltpu.make_async_copy(t_v, out_hbm.at[0, :, pl.ds(b0, BLK)],
                                  osem).wait()

        def issue(h):
            return pltpu.async_copy(table_hbm.at[idxq_v.at[h]],
                                    rows_v.at[h & 3], gsem)

        def drain():
            pltpu.make_async_copy(table_hbm.at[idxq_v.at[0]],
                                  rows_v.at[0], gsem).wait()

        issue(0)
        issue(1)

        def body(c, _):
            h0 = 2 * c

            @pl.when(h0 + 2 < hist)
            def _():
                issue(h0 + 2)

            drain()

            @pl.when(h0 >= 2)
            def _():
                drain_out(t_a)

            select_and_store(rows_v.at[h0 & 3], t_a, h0)

            @pl.when(h0 + 3 < hist)
            def _():
                issue(h0 + 3)

            drain()

            @pl.when(h0 >= 2)
            def _():
                drain_out(t_b)

            select_and_store(rows_v.at[(h0 + 1) & 3], t_b, h0 + 1)
            return 0

        lax.fori_loop(0, hist // 2, body, 0)
        drain_out(t_a)
        drain_out(t_b)

    def repack_body(src_ref, dst_ref):
        x = src_ref[...]
        acc = None
        for k in range(PACK):
            sel = jnp.pad(
                jnp.eye(embed_dim, dtype=jnp.float32),
                ((0, 0), (k * embed_dim, (PACK - 1 - k) * embed_dim)),
            )
            part = jax.lax.dot_general(
                x[:, k * SUB:(k + 1) * SUB], sel,
                (((0,), (0,)), ((), ())),
                preferred_element_type=jnp.float32,
            )
            acc = part if acc is None else acc + part
        dst_ref[...] = acc

    repack = pl.pallas_call(
        repack_body,
        grid=(n_grp,),
        in_specs=[pl.BlockSpec((embed_dim, GRP), lambda j: (0, j))],
        out_specs=pl.BlockSpec((SUB, PACK * embed_dim), lambda j: (j, 0)),
        out_shape=jax.ShapeDtypeStruct((n_grp * SUB, PACK * embed_dim),
                                       jnp.float32),
        compiler_params=pltpu.CompilerParams(
            fuse_transposed_lhs_in_matmul=True
        ),
    )

    def run(x, raw_weight):
        xt = x.T.astype(jnp.int32)
        tq = repack(raw_weight.T)
        out_t = gather_kernel(xt, tq)
        return jnp.transpose(out_t, (2, 0, 1))

    return run


def kernel(x, raw_weight):
    batch, hist = x.shape
    vocab, embed_dim = raw_weight.shape
    return _build(batch, hist, embed_dim, vocab)(x, raw_weight)


# speedup vs baseline: 1.0591x; 1.0459x over previous
import functools

import jax
import jax.numpy as jnp
from jax import lax
from jax.experimental import pallas as pl
from jax.experimental.pallas import tpu as pltpu
from jax.experimental.pallas import tpu_sc as plsc

BLK = 128
PACK = 4
SUB_LOG = 13
SUB = 1 << SUB_LOG
GRP_LOG = SUB_LOG + 2
GRP = 1 << GRP_LOG


@functools.lru_cache(maxsize=None)
def _build(batch, hist, embed_dim, vocab):
    info = plsc.get_sparse_core_info()
    nc, ns = info.num_cores, info.num_subcores
    nw = nc * ns
    assert batch % BLK == 0 and (batch // BLK) == nw
    assert embed_dim == 32 and PACK * embed_dim == 128
    assert hist % 2 == 0
    n_grp = -(-vocab // GRP)

    mesh = plsc.VectorSubcoreMesh(core_axis_name="c", subcore_axis_name="s")

    @functools.partial(
        pl.kernel,
        mesh=mesh,
        out_type=jax.ShapeDtypeStruct((hist, embed_dim, batch), jnp.float32),
        scratch_types=[
            pltpu.VMEM((hist, BLK), jnp.int32),
            pltpu.VMEM((hist, BLK), jnp.int32),
            pltpu.VMEM((hist, BLK), jnp.int32),
            pltpu.VMEM((4, BLK, 128), jnp.float32),
            pltpu.VMEM((embed_dim, BLK), jnp.float32),
            pltpu.VMEM((embed_dim, BLK), jnp.float32),
            pltpu.SemaphoreType.DMA,
            pltpu.SemaphoreType.DMA,
        ],
        compiler_params=pltpu.CompilerParams(
            use_tc_tiling_on_sc=True, needs_layout_passes=False
        ),
    )
    def gather_kernel(xt_hbm, table_hbm, out_hbm, idx_v, idxq_v, colb_v,
                      rows_v, t_a, t_b, gsem, osem):
        wid = lax.axis_index("s") * nc + lax.axis_index("c")
        b0 = wid * BLK
        pltpu.sync_copy(xt_hbm.at[:, pl.ds(b0, BLK)], idx_v)

        lane = lax.iota(jnp.int32, 16)

        @plsc.parallel_loop(0, hist)
        def _(h):
            for bq in range(BLK // 16):
                v = idx_v[h, pl.ds(bq * 16, 16)]
                r = lax.shift_right_logical(v, GRP_LOG)
                idxq_v[h, pl.ds(bq * 16, 16)] = (
                    lax.shift_left(r, SUB_LOG) | (v & (SUB - 1))
                )
                colb_v[h, pl.ds(bq * 16, 16)] = (
                    (lax.shift_right_logical(v, SUB_LOG) & (PACK - 1))
                    * embed_dim
                )

        def select_and_store(rows_ref, t_v, h):
            cbs = [colb_v[h, pl.ds(bq * 16, 16)] for bq in range(BLK // 16)]
            rs = [lane + (bq * 16) for bq in range(BLK // 16)]

            @plsc.parallel_loop(0, embed_dim, unroll=4)
            def _(e):
                for bq in range(BLK // 16):
                    t_v[e, pl.ds(bq * 16, 16)] = plsc.load_gather(
                        rows_ref, [rs[bq], cbs[bq] + e]
                    )

            pltpu.async_copy(t_v, out_hbm.at[h, :, pl.ds(b0, BLK)], osem)

        def drain_out(t_v):
            pltpu.make_async_copy(t_v, out_hbm.at[0, :, pl.ds(b0, BLK)],
                                  osem).wait()

        def issue(h):
            return pltpu.async_copy(table_hbm.at[idxq_v.at[h]],
                                    rows_v.at[h & 3], gsem)

        def drain():
            pltpu.make_async_copy(table_hbm.at[idxq_v.at[0]],
                                  rows_v.at[0], gsem).wait()

        issue(0)
        issue(1)

        def body(c, _):
            h0 = 2 * c

            @pl.when(h0 + 2 < hist)
            def _():
                issue(h0 + 2)

            drain()

            @pl.when(h0 >= 2)
            def _():
                drain_out(t_a)

            select_and_store(rows_v.at[h0 & 3], t_a, h0)

            @pl.when(h0 + 3 < hist)
            def _():
                issue(h0 + 3)

            drain()

            @pl.when(h0 >= 2)
            def _():
                drain_out(t_b)

            select_and_store(rows_v.at[(h0 + 1) & 3], t_b, h0 + 1)
            return 0

        lax.fori_loop(0, hist // 2, body, 0)
        drain_out(t_a)
        drain_out(t_b)

    def repack_body(src_ref, dst_ref):
        x = src_ref[...]
        acc = None
        for k in range(PACK):
            sel = jnp.pad(
                jnp.eye(embed_dim, dtype=jnp.float32),
                ((0, 0), (k * embed_dim, (PACK - 1 - k) * embed_dim)),
            )
            part = jax.lax.dot_general(
                x[:, k * SUB:(k + 1) * SUB], sel,
                (((0,), (0,)), ((), ())),
                preferred_element_type=jnp.float32,
            )
            acc = part if acc is None else acc + part
        dst_ref[...] = acc

    repack = pl.pallas_call(
        repack_body,
        grid=(n_grp,),
        in_specs=[pl.BlockSpec((embed_dim, GRP), lambda j: (0, j))],
        out_specs=pl.BlockSpec((SUB, PACK * embed_dim), lambda j: (j, 0)),
        out_shape=jax.ShapeDtypeStruct((n_grp * SUB, PACK * embed_dim),
                                       jnp.float32),
        compiler_params=pltpu.CompilerParams(
            fuse_transposed_lhs_in_matmul=True
        ),
    )

    def run(x, raw_weight):
        xt = x.T.astype(jnp.int32)
        tq = repack(raw_weight.T)
        out_t = gather_kernel(xt, tq)
        return jnp.transpose(out_t, (2, 0, 1))

    return run


def kernel(x, raw_weight):
    batch, hist = x.shape
    vocab, embed_dim = raw_weight.shape
    return _build(batch, hist, embed_dim, vocab)(x, raw_weight)
